# Initial kernel scaffold; baseline (speedup 1.0000x reference)
import jax
import jax.numpy as jnp
from jax.experimental import pallas as pl


def kernel(actions, embedding):
    raise NotImplementedError("write your pallas kernel here")


# speedup vs baseline: 3.3710x; 3.3710x over previous
import functools

import jax
import jax.numpy as jnp
from jax import lax
from jax.experimental import pallas as pl
from jax.experimental.pallas import tpu as pltpu
from jax.experimental.pallas import tpu_sc as plsc

NC = 2
NS = 16
NW = NC * NS

LANES = 16
CHUNK = 128
ROUND_MAGIC = 8388608.0


@functools.lru_cache(maxsize=None)
def _build(n_total, vocab, embed_dim):
    n_per_w = n_total // NW
    n_chunks = n_per_w // CHUNK
    assert n_per_w * NW == n_total
    assert n_chunks * CHUNK == n_per_w
    assert n_chunks >= 2 and n_chunks % 2 == 0
    mesh = plsc.VectorSubcoreMesh(core_axis_name="c", subcore_axis_name="s")

    @functools.partial(
        pl.kernel,
        mesh=mesh,
        compiler_params=pltpu.CompilerParams(use_tc_tiling_on_sc=False),
        out_type=jax.ShapeDtypeStruct((n_total, embed_dim), jnp.float32),
        scratch_types=[
            pltpu.VMEM((n_per_w,), jnp.float32),
            pltpu.VMEM((n_per_w,), jnp.int32),
            pltpu.VMEM((2, CHUNK, embed_dim), jnp.float32),
            pltpu.SemaphoreType.DMA,
            pltpu.SemaphoreType.DMA,
        ],
    )
    def k(act_hbm, table_hbm, out_hbm, act_v, idx_v, rows_v, sem_g, sem_s):
        wid = lax.axis_index("s") * NC + lax.axis_index("c")
        base = wid * n_per_w
        pltpu.sync_copy(act_hbm.at[pl.ds(base, n_per_w)], act_v)

        def qbody(i, _):
            x = act_v[pl.ds(i * LANES, LANES)]
            s = (x - (-1.0)) / 2.0 * (vocab - 1.0)
            t = (s + ROUND_MAGIC) - ROUND_MAGIC
            t = jnp.minimum(jnp.maximum(t, 0.0), vocab - 1.0)
            idx_v[pl.ds(i * LANES, LANES)] = t.astype(jnp.int32)
            return 0

        lax.fori_loop(0, n_per_w // LANES, qbody, 0, unroll=4)

        def gather(j, buf):
            return pltpu.async_copy(
                table_hbm.at[idx_v.at[pl.ds(j * CHUNK, CHUNK)]],
                rows_v.at[buf], sem_g)

        def scatter(j, buf):
            pltpu.async_copy(
                rows_v.at[buf], out_hbm.at[pl.ds(base + j * CHUNK, CHUNK)],
                sem_s)

        def wait_one_scatter():
            pltpu.make_async_copy(
                rows_v.at[0], out_hbm.at[pl.ds(base, CHUNK)], sem_s).wait()

        gather(0, 0).wait()
        scatter(0, 0)
        gather(1, 1).wait()
        scatter(1, 1)

        def body(i, _):
            j = 2 + 2 * i
            wait_one_scatter()
            gather(j, 0).wait()
            scatter(j, 0)
            wait_one_scatter()
            gather(j + 1, 1).wait()
            scatter(j + 1, 1)
            return 0

        lax.fori_loop(0, (n_chunks - 2) // 2, body, 0)
        wait_one_scatter()
        wait_one_scatter()

    return k


def kernel(actions, embedding):
    b, t, a = actions.shape
    vocab, embed_dim = embedding.shape
    n_total = b * t * a
    out = _build(n_total, vocab, embed_dim)(
        actions.reshape(n_total), embedding)
    return out.reshape(b, t, a, embed_dim)


# speedup vs baseline: 3.4388x; 1.0201x over previous
import functools

import jax
import jax.numpy as jnp
from jax import lax
from jax.experimental import pallas as pl
from jax.experimental.pallas import tpu as pltpu
from jax.experimental.pallas import tpu_sc as plsc

NC = 2
NS = 16
NW = NC * NS

LANES = 16
CHUNK = 128
GPC = 4
SUPER = CHUNK * GPC
ROUND_MAGIC = 8388608.0


@functools.lru_cache(maxsize=None)
def _build(n_total, vocab, embed_dim):
    n_per_w = n_total // NW
    n_super = n_per_w // SUPER
    assert n_per_w * NW == n_total
    assert n_super * SUPER == n_per_w
    assert n_super >= 2 and n_super % 2 == 0
    mesh = plsc.VectorSubcoreMesh(core_axis_name="c", subcore_axis_name="s")

    @functools.partial(
        pl.kernel,
        mesh=mesh,
        compiler_params=pltpu.CompilerParams(use_tc_tiling_on_sc=False),
        out_type=jax.ShapeDtypeStruct((n_total, embed_dim), jnp.float32),
        scratch_types=[
            pltpu.VMEM((n_per_w,), jnp.float32),
            pltpu.VMEM((n_per_w,), jnp.int32),
            pltpu.VMEM((2, SUPER, embed_dim), jnp.float32),
            pltpu.SemaphoreType.DMA,
            pltpu.SemaphoreType.DMA,
        ],
    )
    def k(act_hbm, table_hbm, out_hbm, act_v, idx_v, rows_v, sem_g, sem_s):
        wid = lax.axis_index("s") * NC + lax.axis_index("c")
        base = wid * n_per_w
        pltpu.sync_copy(act_hbm.at[pl.ds(base, n_per_w)], act_v)

        def qbody(i, _):
            x = act_v[pl.ds(i * LANES, LANES)]
            s = (x - (-1.0)) / 2.0 * (vocab - 1.0)
            t = (s + ROUND_MAGIC) - ROUND_MAGIC
            t = jnp.minimum(jnp.maximum(t, 0.0), vocab - 1.0)
            idx_v[pl.ds(i * LANES, LANES)] = t.astype(jnp.int32)
            return 0

        lax.fori_loop(0, n_per_w // LANES, qbody, 0, unroll=4)

        def gather_super(j, buf):
            cps = []
            for k in range(GPC):
                cps.append(pltpu.async_copy(
                    table_hbm.at[
                        idx_v.at[pl.ds(j * SUPER + k * CHUNK, CHUNK)]],
                    rows_v.at[buf].at[pl.ds(k * CHUNK, CHUNK)], sem_g))
            return cps

        def drain_gathers(cps):
            for cp in cps:
                cp.wait()

        def scatter(j, buf):
            pltpu.async_copy(
                rows_v.at[buf], out_hbm.at[pl.ds(base + j * SUPER, SUPER)],
                sem_s)

        def wait_one_scatter():
            pltpu.make_async_copy(
                rows_v.at[0], out_hbm.at[pl.ds(base, SUPER)], sem_s).wait()

        drain_gathers(gather_super(0, 0))
        scatter(0, 0)
        drain_gathers(gather_super(1, 1))
        scatter(1, 1)

        def body(i, _):
            j = 2 + 2 * i
            wait_one_scatter()
            drain_gathers(gather_super(j, 0))
            scatter(j, 0)
            wait_one_scatter()
            drain_gathers(gather_super(j + 1, 1))
            scatter(j + 1, 1)
            return 0

        lax.fori_loop(0, (n_super - 2) // 2, body, 0)
        wait_one_scatter()
        wait_one_scatter()

    return k


def kernel(actions, embedding):
    b, t, a = actions.shape
    vocab, embed_dim = embedding.shape
    n_total = b * t * a
    out = _build(n_total, vocab, embed_dim)(
        actions.reshape(n_total), embedding)
    return out.reshape(b, t, a, embed_dim)


# speedup vs baseline: 3.5030x; 1.0187x over previous
import functools

import jax
import jax.numpy as jnp
from jax import lax
from jax.experimental import pallas as pl
from jax.experimental.pallas import tpu as pltpu
from jax.experimental.pallas import tpu_sc as plsc

NC = 2
NS = 16
NW = NC * NS

LANES = 16
CHUNK = 128
GPC = 2
SUPER = CHUNK * GPC
NBUF = 4
ROUND_MAGIC = 8388608.0


@functools.lru_cache(maxsize=None)
def _build(n_total, vocab, embed_dim):
    n_per_w = n_total // NW
    n_super = n_per_w // SUPER
    assert n_per_w * NW == n_total
    assert n_super * SUPER == n_per_w
    assert n_super >= NBUF and (n_super - NBUF) % NBUF == 0
    mesh = plsc.VectorSubcoreMesh(core_axis_name="c", subcore_axis_name="s")

    @functools.partial(
        pl.kernel,
        mesh=mesh,
        compiler_params=pltpu.CompilerParams(use_tc_tiling_on_sc=False),
        out_type=jax.ShapeDtypeStruct((n_total, embed_dim), jnp.float32),
        scratch_types=[
            pltpu.VMEM((n_per_w,), jnp.float32),
            pltpu.VMEM((n_per_w,), jnp.int32),
            pltpu.VMEM((NBUF, SUPER, embed_dim), jnp.float32),
            pltpu.SemaphoreType.DMA,
            pltpu.SemaphoreType.DMA,
        ],
    )
    def k(act_hbm, table_hbm, out_hbm, act_v, idx_v, rows_v, sem_g, sem_s):
        wid = lax.axis_index("s") * NC + lax.axis_index("c")
        base = wid * n_per_w
        pltpu.sync_copy(act_hbm.at[pl.ds(base, n_per_w)], act_v)

        def fire(j, buf):
            for i in range(SUPER // LANES):
                x = act_v[pl.ds(j * SUPER + i * LANES, LANES)]
                s = (x - (-1.0)) / 2.0 * (vocab - 1.0)
                t = (s + ROUND_MAGIC) - ROUND_MAGIC
                t = jnp.minimum(jnp.maximum(t, 0.0), vocab - 1.0)
                idx_v[pl.ds(j * SUPER + i * LANES, LANES)] = \
                    t.astype(jnp.int32)
            for k in range(GPC):
                pltpu.async_copy(
                    table_hbm.at[
                        idx_v.at[pl.ds(j * SUPER + k * CHUNK, CHUNK)]],
                    rows_v.at[buf].at[pl.ds(k * CHUNK, CHUNK)], sem_g)

        def wait_gathers():
            pltpu.make_async_copy(
                table_hbm.at[pl.ds(0, SUPER)], rows_v.at[0], sem_g).wait()

        def scatter(j, buf):
            pltpu.async_copy(
                rows_v.at[buf], out_hbm.at[pl.ds(base + j * SUPER, SUPER)],
                sem_s)

        def wait_one_scatter():
            pltpu.make_async_copy(
                rows_v.at[0], out_hbm.at[pl.ds(base, SUPER)], sem_s).wait()

        for j in range(NBUF - 1):
            fire(j, j)

        wait_gathers()
        scatter(0, 0)
        fire(NBUF - 1, NBUF - 1)

        def body(i, _):
            for r in range(NBUF):
                j = 1 + NBUF * i + r
                wait_gathers()
                scatter(j, (1 + r) % NBUF)
                wait_one_scatter()
                fire(j + NBUF - 1, r)
            return 0

        lax.fori_loop(0, (n_super - NBUF) // NBUF, body, 0)

        for j in range(n_super - NBUF + 1, n_super):
            wait_gathers()
            scatter(j, j % NBUF)
            wait_one_scatter()
        wait_one_scatter()

    return k


def kernel(actions, embedding):
    b, t, a = actions.shape
    vocab, embed_dim = embedding.shape
    n_total = b * t * a
    out = _build(n_total, vocab, embed_dim)(
        actions.reshape(n_total), embedding)
    return out.reshape(b, t, a, embed_dim)
